# Initial kernel scaffold; baseline (speedup 1.0000x reference)
import jax
import jax.numpy as jnp
from jax.experimental import pallas as pl


def kernel(atom_types, edge_index, edge_types, batch_idx, atom_emb, in_W, in_b, edge_emb, msg_W1, msg_b1, msg_W2, msg_b2, upd_W1, upd_b1, upd_W2, upd_b2, ln_g, ln_b, out_W, out_b):
    raise NotImplementedError("write your pallas kernel here")


# speedup vs baseline: 1.0883x; 1.0883x over previous
import functools

import jax
import jax.numpy as jnp
from jax import lax
from jax.experimental import pallas as pl
from jax.experimental.pallas import tpu as pltpu


def _gelu(x):
    return 0.5 * x * (1.0 + lax.erf(x * 0.7071067811865476))


def _pq_body(x_ref, wa_ref, wb_ref, p_ref, q_ref):
    x = x_ref[...]
    p_ref[...] = jnp.dot(x, wa_ref[...], preferred_element_type=jnp.float32)
    q_ref[...] = jnp.dot(x, wb_ref[...], preferred_element_type=jnp.float32)


def _pq(x, wa, wb, blk):
    n = x.shape[0]
    h2 = wa.shape[1]
    grid = (n // blk,)
    return pl.pallas_call(
        _pq_body,
        grid=grid,
        in_specs=[
            pl.BlockSpec((blk, x.shape[1]), lambda i: (i, 0)),
            pl.BlockSpec(wa.shape, lambda i: (0, 0)),
            pl.BlockSpec(wb.shape, lambda i: (0, 0)),
        ],
        out_specs=[
            pl.BlockSpec((blk, h2), lambda i: (i, 0)),
            pl.BlockSpec((blk, h2), lambda i: (i, 0)),
        ],
        out_shape=[
            jax.ShapeDtypeStruct((n, h2), jnp.float32),
            jax.ShapeDtypeStruct((n, h2), jnp.float32),
        ],
    )(x, wa, wb)


def _edge_body(pre_ref, w2_ref, b2_ref, out_ref):
    m = _gelu(pre_ref[...])
    out_ref[...] = (
        jnp.dot(m, w2_ref[...], preferred_element_type=jnp.float32) + b2_ref[...]
    )


def _edge(pre, w2, b2, blk):
    e, h2 = pre.shape
    h = w2.shape[1]
    return pl.pallas_call(
        _edge_body,
        grid=(e // blk,),
        in_specs=[
            pl.BlockSpec((blk, h2), lambda i: (i, 0)),
            pl.BlockSpec(w2.shape, lambda i: (0, 0)),
            pl.BlockSpec((1, h), lambda i: (0, 0)),
        ],
        out_specs=pl.BlockSpec((blk, h), lambda i: (i, 0)),
        out_shape=jax.ShapeDtypeStruct((e, h), jnp.float32),
    )(pre, w2, b2.reshape(1, h))


def _upd_body(x_ref, a_ref, wa_ref, wb_ref, b1_ref, w2_ref, b2_ref, g_ref,
              bb_ref, out_ref):
    x = x_ref[...]
    h = jnp.dot(x, wa_ref[...], preferred_element_type=jnp.float32)
    h += jnp.dot(a_ref[...], wb_ref[...], preferred_element_type=jnp.float32)
    h = _gelu(h + b1_ref[...])
    u = jnp.dot(h, w2_ref[...], preferred_element_type=jnp.float32) + b2_ref[...]
    y = x + u
    m = jnp.mean(y, axis=-1, keepdims=True)
    v = jnp.mean((y - m) ** 2, axis=-1, keepdims=True)
    out_ref[...] = (y - m) * lax.rsqrt(v + 1e-5) * g_ref[...] + bb_ref[...]


def _update(x, aggr, wa, wb, b1, w2, b2, g, bb, blk):
    n, h = x.shape
    row = lambda a: a.reshape(1, -1)
    return pl.pallas_call(
        _upd_body,
        grid=(n // blk,),
        in_specs=[
            pl.BlockSpec((blk, h), lambda i: (i, 0)),
            pl.BlockSpec((blk, h), lambda i: (i, 0)),
            pl.BlockSpec(wa.shape, lambda i: (0, 0)),
            pl.BlockSpec(wb.shape, lambda i: (0, 0)),
            pl.BlockSpec((1, h), lambda i: (0, 0)),
            pl.BlockSpec(w2.shape, lambda i: (0, 0)),
            pl.BlockSpec((1, h), lambda i: (0, 0)),
            pl.BlockSpec((1, h), lambda i: (0, 0)),
            pl.BlockSpec((1, h), lambda i: (0, 0)),
        ],
        out_specs=pl.BlockSpec((blk, h), lambda i: (i, 0)),
        out_shape=jax.ShapeDtypeStruct((n, h), jnp.float32),
    )(x, aggr, wa, wb, row(b1), w2, row(b2), row(g), row(bb))


def _mm_body(x_ref, w_ref, b_ref, out_ref):
    out_ref[...] = (
        jnp.dot(x_ref[...], w_ref[...], preferred_element_type=jnp.float32)
        + b_ref[...]
    )


def _mm(x, w, b):
    n = x.shape[0]
    o = w.shape[1]
    return pl.pallas_call(
        _mm_body,
        in_specs=[
            pl.BlockSpec(x.shape, lambda: (0, 0)),
            pl.BlockSpec(w.shape, lambda: (0, 0)),
            pl.BlockSpec((1, o), lambda: (0, 0)),
        ],
        out_specs=pl.BlockSpec((n, o), lambda: (0, 0)),
        out_shape=jax.ShapeDtypeStruct((n, o), jnp.float32),
    )(x, w, b.reshape(1, o))


def kernel(atom_types, edge_index, edge_types, batch_idx, atom_emb, in_W,
           in_b, edge_emb, msg_W1, msg_b1, msg_W2, msg_b2, upd_W1, upd_b1,
           upd_W2, upd_b2, ln_g, ln_b, out_W, out_b):
    n = atom_types.shape[0]
    nlayers, _, h = msg_W2.shape
    nbatch = 256
    row, col = edge_index[0], edge_index[1]

    tbl = _mm(atom_emb, in_W, in_b)
    x = jnp.take(tbl, atom_types, axis=0)

    cnt = jnp.zeros((n, 1), jnp.float32).at[col].add(1.0)
    inv = 1.0 / jnp.clip(cnt, 1.0, None)

    for l in range(nlayers):
        w1 = msg_W1[l]
        wa, wb, wc = w1[:h], w1[h:2 * h], w1[2 * h:]
        p, q = _pq(x, wa, wb, 1000)
        rt = _mm(edge_emb, wc, msg_b1[l])
        pre = (jnp.take(p, row, axis=0) + jnp.take(q, col, axis=0)
               + jnp.take(rt, edge_types, axis=0))
        msg = _edge(pre, msg_W2[l], msg_b2[l], 1000)
        aggr = jnp.zeros((n, h), jnp.float32).at[col].add(msg) * inv
        uw = upd_W1[l]
        x = _update(x, aggr, uw[:h], uw[h:], upd_b1[l], upd_W2[l],
                    upd_b2[l], ln_g[l], ln_b[l], 1000)

    pooled = jnp.zeros((nbatch, h), jnp.float32).at[batch_idx].add(x)
    bcnt = jnp.zeros((nbatch, 1), jnp.float32).at[batch_idx].add(1.0)
    pooled = pooled / jnp.clip(bcnt, 1.0, None)
    return _mm(pooled, out_W, out_b)


# speedup vs baseline: 1.0900x; 1.0016x over previous
import functools

import jax
import jax.numpy as jnp
from jax import lax
from jax.experimental import pallas as pl
from jax.experimental.pallas import tpu as pltpu


def _gelu(x):
    return 0.5 * x * (1.0 + lax.erf(x * 0.7071067811865476))


def _pq_body(x_ref, wa_ref, wb_ref, p_ref, q_ref):
    x = x_ref[...].astype(jnp.bfloat16)
    p_ref[...] = jnp.dot(x, wa_ref[...], preferred_element_type=jnp.float32)
    q_ref[...] = jnp.dot(x, wb_ref[...], preferred_element_type=jnp.float32)


def _pq(x, wa, wb, blk):
    n = x.shape[0]
    h2 = wa.shape[1]
    grid = (n // blk,)
    return pl.pallas_call(
        _pq_body,
        grid=grid,
        in_specs=[
            pl.BlockSpec((blk, x.shape[1]), lambda i: (i, 0)),
            pl.BlockSpec(wa.shape, lambda i: (0, 0)),
            pl.BlockSpec(wb.shape, lambda i: (0, 0)),
        ],
        out_specs=[
            pl.BlockSpec((blk, h2), lambda i: (i, 0)),
            pl.BlockSpec((blk, h2), lambda i: (i, 0)),
        ],
        out_shape=[
            jax.ShapeDtypeStruct((n, h2), jnp.float32),
            jax.ShapeDtypeStruct((n, h2), jnp.float32),
        ],
    )(x, wa, wb)


def _edge_body(pre_ref, w2_ref, b2_ref, out_ref):
    m = _gelu(pre_ref[...]).astype(jnp.bfloat16)
    out_ref[...] = (
        jnp.dot(m, w2_ref[...], preferred_element_type=jnp.float32) + b2_ref[...]
    )


def _edge(pre, w2, b2, blk):
    e, h2 = pre.shape
    h = w2.shape[1]
    return pl.pallas_call(
        _edge_body,
        grid=(e // blk,),
        in_specs=[
            pl.BlockSpec((blk, h2), lambda i: (i, 0)),
            pl.BlockSpec(w2.shape, lambda i: (0, 0)),
            pl.BlockSpec((1, h), lambda i: (0, 0)),
        ],
        out_specs=pl.BlockSpec((blk, h), lambda i: (i, 0)),
        out_shape=jax.ShapeDtypeStruct((e, h), jnp.float32),
    )(pre, w2, b2.reshape(1, h))


def _upd_body(x_ref, a_ref, wa_ref, wb_ref, b1_ref, w2_ref, b2_ref, g_ref,
              bb_ref, out_ref):
    x = x_ref[...]
    h = jnp.dot(x.astype(jnp.bfloat16), wa_ref[...],
                preferred_element_type=jnp.float32)
    h += jnp.dot(a_ref[...].astype(jnp.bfloat16), wb_ref[...],
                 preferred_element_type=jnp.float32)
    h = _gelu(h + b1_ref[...]).astype(jnp.bfloat16)
    u = jnp.dot(h, w2_ref[...], preferred_element_type=jnp.float32) + b2_ref[...]
    y = x + u
    m = jnp.mean(y, axis=-1, keepdims=True)
    v = jnp.mean((y - m) ** 2, axis=-1, keepdims=True)
    out_ref[...] = (y - m) * lax.rsqrt(v + 1e-5) * g_ref[...] + bb_ref[...]


def _update(x, aggr, wa, wb, b1, w2, b2, g, bb, blk):
    n, h = x.shape
    row = lambda a: a.reshape(1, -1)
    return pl.pallas_call(
        _upd_body,
        grid=(n // blk,),
        in_specs=[
            pl.BlockSpec((blk, h), lambda i: (i, 0)),
            pl.BlockSpec((blk, h), lambda i: (i, 0)),
            pl.BlockSpec(wa.shape, lambda i: (0, 0)),
            pl.BlockSpec(wb.shape, lambda i: (0, 0)),
            pl.BlockSpec((1, h), lambda i: (0, 0)),
            pl.BlockSpec(w2.shape, lambda i: (0, 0)),
            pl.BlockSpec((1, h), lambda i: (0, 0)),
            pl.BlockSpec((1, h), lambda i: (0, 0)),
            pl.BlockSpec((1, h), lambda i: (0, 0)),
        ],
        out_specs=pl.BlockSpec((blk, h), lambda i: (i, 0)),
        out_shape=jax.ShapeDtypeStruct((n, h), jnp.float32),
    )(x, aggr, wa, wb, row(b1), w2, row(b2), row(g), row(bb))


def _mm_body(x_ref, w_ref, b_ref, out_ref):
    out_ref[...] = (
        jnp.dot(x_ref[...], w_ref[...], preferred_element_type=jnp.float32)
        + b_ref[...]
    )


def _mm(x, w, b):
    n = x.shape[0]
    o = w.shape[1]
    return pl.pallas_call(
        _mm_body,
        in_specs=[
            pl.BlockSpec(x.shape, lambda: (0, 0)),
            pl.BlockSpec(w.shape, lambda: (0, 0)),
            pl.BlockSpec((1, o), lambda: (0, 0)),
        ],
        out_specs=pl.BlockSpec((n, o), lambda: (0, 0)),
        out_shape=jax.ShapeDtypeStruct((n, o), jnp.float32),
    )(x, w, b.reshape(1, o))


def kernel(atom_types, edge_index, edge_types, batch_idx, atom_emb, in_W,
           in_b, edge_emb, msg_W1, msg_b1, msg_W2, msg_b2, upd_W1, upd_b1,
           upd_W2, upd_b2, ln_g, ln_b, out_W, out_b):
    n = atom_types.shape[0]
    nlayers, _, h = msg_W2.shape
    nbatch = 256
    row, col = edge_index[0], edge_index[1]

    tbl = _mm(atom_emb, in_W, in_b)
    x = jnp.take(tbl, atom_types, axis=0)

    cnt = jnp.zeros((n, 1), jnp.float32).at[col].add(1.0)
    inv = 1.0 / jnp.clip(cnt, 1.0, None)

    bf = jnp.bfloat16
    for l in range(nlayers):
        w1 = msg_W1[l]
        wa, wb, wc = w1[:h].astype(bf), w1[h:2 * h].astype(bf), w1[2 * h:]
        p, q = _pq(x, wa, wb, 1000)
        rt = _mm(edge_emb, wc, msg_b1[l])
        pre = (jnp.take(p, row, axis=0) + jnp.take(q, col, axis=0)
               + jnp.take(rt, edge_types, axis=0))
        msg = _edge(pre, msg_W2[l].astype(bf), msg_b2[l], 1000)
        aggr = jnp.zeros((n, h), jnp.float32).at[col].add(msg) * inv
        uw = upd_W1[l]
        x = _update(x, aggr, uw[:h].astype(bf), uw[h:].astype(bf),
                    upd_b1[l], upd_W2[l].astype(bf),
                    upd_b2[l], ln_g[l], ln_b[l], 1000)

    pooled = jnp.zeros((nbatch, h), jnp.float32).at[batch_idx].add(x)
    bcnt = jnp.zeros((nbatch, 1), jnp.float32).at[batch_idx].add(1.0)
    pooled = pooled / jnp.clip(bcnt, 1.0, None)
    return _mm(pooled, out_W, out_b)
